# baseline (speedup 1.0000x reference)
import functools

import jax
import jax.numpy as jnp
from jax import lax
from jax.experimental import pallas as pl
from jax.experimental.pallas import tpu as pltpu
from jax.experimental.pallas import tpu_sc as plsc

_NC = 2
_NS = 16
_EB = 128
_PAD = 128


def _elu(x):
    return jnp.where(x > 0, x, jnp.exp(jnp.minimum(x, 0.0)) - 1.0)


def _num_batches(e):
    nb = -(-e // (_NC * _NS * _EB))
    return -(-nb // 4) * 4


def _make_seg_sum(n, e, d, num_chunks, with_deg):
    assert d % 128 == 0
    nb = _num_batches(e)
    na = n + _PAD
    rbase = (n // (8 * _NS)) * 8
    zrem = na - _NS * rbase
    frem = n - _NS * rbase
    assert zrem % 8 == 0 and frem % 8 == 0 and frem >= 0

    nh = nb // 2
    mesh = plsc.VectorSubcoreMesh(core_axis_name="c", subcore_axis_name="s")
    out_type = [jax.ShapeDtypeStruct((_NC, n, d), jnp.float32)
                for _ in range(num_chunks + int(with_deg))]
    scratch = [
        pltpu.VMEM((nh, _EB), jnp.int32),
        pltpu.VMEM((nh, _EB), jnp.int32),
        pltpu.VMEM((_EB, d), jnp.float32),
        pltpu.VMEM((_EB, d), jnp.float32),
        pltpu.VMEM_SHARED((na, d), jnp.float32),
        pltpu.SemaphoreType.DMA,
        pltpu.SemaphoreType.DMA,
    ]

    @functools.partial(pl.kernel, out_type=out_type, mesh=mesh,
                       scratch_types=scratch)
    def k(*refs):
        it = iter(refs)
        tables = [next(it) for _ in range(num_chunks)]
        src_hbm, dst_hbm, zd_hbm = next(it), next(it), next(it)
        ones_hbm = next(it) if with_deg else None
        outs = [next(it) for _ in range(num_chunks + int(with_deg))]
        src_v, dst_v, rows0, rows1, acc_sh, sem0, sem1 = (
            next(it) for _ in range(7))

        c = lax.axis_index("c")
        s = lax.axis_index("s")
        wid = s * _NC + c
        row0 = pl.multiple_of(s * rbase, 8)

        def copy_rows(rem, get_src_dst):
            src, dst = get_src_dst(row0, rbase)
            pltpu.sync_copy(src, dst)
            if rem:
                @pl.when(s == _NS - 1)
                def _():
                    srct, dstt = get_src_dst(_NS * rbase, rem)
                    pltpu.sync_copy(srct, dstt)

        def load_idx(half):
            pltpu.sync_copy(src_hbm.at[wid, pl.ds(half * nh, nh)], src_v)
            pltpu.sync_copy(dst_hbm.at[wid, pl.ds(half * nh, nh)], dst_v)

        for ci in range(num_chunks + int(with_deg)):
            deg_pass = ci == num_chunks
            copy_rows(zrem, lambda r, m: (zd_hbm.at[pl.ds(r, m)],
                                          acc_sh.at[pl.ds(r, m)]))
            if deg_pass:
                pltpu.sync_copy(ones_hbm, rows0)
            plsc.subcore_barrier()

            for half in range(2):
                load_idx(half)
                if deg_pass:
                    def dbody(j, carry):
                        pltpu.async_copy(rows0, acc_sh.at[dst_v.at[j]],
                                         sem0, add=True)
                        return carry

                    lax.fori_loop(0, nh, dbody, 0)

                    def dwait(j, carry):
                        pltpu.make_async_copy(
                            rows0, acc_sh.at[dst_v.at[0]], sem0).wait()
                        return carry

                    lax.fori_loop(0, nh, dwait, 0)
                else:
                    tab = tables[ci]

                    def gbody(j, carry):
                        pltpu.async_copy(
                            tab.at[src_v.at[j]], rows0, sem0).wait()
                        pltpu.sync_copy(rows0, acc_sh.at[dst_v.at[j]],
                                        add=True)
                        return carry

                    lax.fori_loop(0, nh, gbody, 0)

            plsc.subcore_barrier()
            out_ref = outs[ci]
            copy_rows(frem, lambda r, m: (acc_sh.at[pl.ds(r, m)],
                                          out_ref.at[c, pl.ds(r, m)]))

    return k


_R = 2000


def _inv_deg(dg_ref):
    deg = dg_ref[0, :, 0:1] + dg_ref[1, :, 0:1]
    return 1.0 / jnp.maximum(deg, 1.0)


def _dot(a, b):
    return jnp.dot(a, b, preferred_element_type=jnp.float32)


def _row_blk(w):
    return pl.BlockSpec((_R, w), lambda i: (i, 0))


def _part_blk(w):
    return pl.BlockSpec((_NC, _R, w), lambda i: (0, i, 0))


def _full(a):
    return pl.BlockSpec(a.shape, lambda i: (0,) * a.ndim)


def _tc_layer1(features, s1, deg16, w_self1, w_neigh1, b1, w_res, b_res,
               w_neigh2):
    n, d_in = features.shape
    h1 = w_self1.shape[1]
    h2 = w_neigh2.shape[1]
    grid = n // _R

    def body(f_ref, s1_ref, dg_ref, ws1_ref, wn1_ref, b1_ref, wr_ref, br_ref,
             wn2_ref, x1_ref, res_ref, n2a_ref, n2b_ref, n2c_ref):
        inv = _inv_deg(dg_ref)
        m1 = (s1_ref[0] + s1_ref[1]) * inv
        f = f_ref[...]
        x1 = _elu(_dot(f, ws1_ref[...]) + _dot(m1, wn1_ref[...]) + b1_ref[...])
        x1_ref[...] = x1
        res_ref[...] = _elu(_dot(f, wr_ref[...]) + br_ref[...])
        n2 = _dot(x1, wn2_ref[...])
        n2a_ref[...] = n2[:, :128]
        n2b_ref[...] = n2[:, 128:256]
        n2c_ref[...] = jnp.concatenate(
            [n2[:, 256:], jnp.zeros((_R, 128 - (h2 - 256)), jnp.float32)],
            axis=1)

    return pl.pallas_call(
        body,
        grid=(grid,),
        in_specs=[_row_blk(d_in), _part_blk(d_in), _part_blk(16),
                  _full(w_self1), _full(w_neigh1), _full(b1), _full(w_res),
                  _full(b_res), _full(w_neigh2)],
        out_specs=[_row_blk(h1), _row_blk(d_in), _row_blk(128), _row_blk(128),
                   _row_blk(128)],
        out_shape=[jax.ShapeDtypeStruct((n, h1), jnp.float32),
                   jax.ShapeDtypeStruct((n, d_in), jnp.float32),
                   jax.ShapeDtypeStruct((n, 128), jnp.float32),
                   jax.ShapeDtypeStruct((n, 128), jnp.float32),
                   jax.ShapeDtypeStruct((n, 128), jnp.float32)],
    )(features, s1, deg16, w_self1, w_neigh1, b1, w_res, b_res, w_neigh2)


def _tc_layer2(x1, s2a, s2b, s2c, deg16, w_self2, b2, w_neigh3):
    n, h1 = x1.shape
    h2 = w_self2.shape[1]
    d_out = w_neigh3.shape[1]
    grid = n // _R

    def body(x1_ref, s2a_ref, s2b_ref, s2c_ref, dg_ref, ws2_ref, b2_ref,
             wn3_ref, x2_ref, n3_ref):
        inv = _inv_deg(dg_ref)
        m2 = jnp.concatenate(
            [(s2a_ref[0] + s2a_ref[1]) * inv,
             (s2b_ref[0] + s2b_ref[1]) * inv,
             ((s2c_ref[0] + s2c_ref[1]) * inv)[:, :h2 - 256]], axis=1)
        x2 = _elu(_dot(x1_ref[...], ws2_ref[...]) + m2 + b2_ref[...])
        x2_ref[...] = x2
        n3_ref[...] = _dot(x2, wn3_ref[...])

    return pl.pallas_call(
        body,
        grid=(grid,),
        in_specs=[_row_blk(h1), _part_blk(128), _part_blk(128),
                  _part_blk(128), _part_blk(16),
                  _full(w_self2), _full(b2), _full(w_neigh3)],
        out_specs=[_row_blk(h2), _row_blk(d_out)],
        out_shape=[jax.ShapeDtypeStruct((n, h2), jnp.float32),
                   jax.ShapeDtypeStruct((n, d_out), jnp.float32)],
    )(x1, s2a, s2b, s2c, deg16, w_self2, b2, w_neigh3)


def _tc_layer3(x2, s3, deg16, w_self3, b3):
    n, h2 = x2.shape
    d_out = w_self3.shape[1]
    grid = n // _R

    def body(x2_ref, s3_ref, dg_ref, ws3_ref, b3_ref, x3_ref):
        inv = _inv_deg(dg_ref)
        m3 = (s3_ref[0] + s3_ref[1]) * inv
        x3_ref[...] = _elu(_dot(x2_ref[...], ws3_ref[...]) + m3 + b3_ref[...])

    return pl.pallas_call(
        body,
        grid=(grid,),
        in_specs=[_row_blk(h2), _part_blk(d_out), _part_blk(16),
                  _full(w_self3), _full(b3)],
        out_specs=[_row_blk(d_out)],
        out_shape=[jax.ShapeDtypeStruct((n, d_out), jnp.float32)],
    )(x2, s3, deg16, w_self3, b3)[0]


def kernel(features, edge_index, W_self1, W_neigh1, b1, W_self2, W_neigh2,
           b2, W_self3, W_neigh3, b3, W_res, b_res):
    n, d_in = features.shape
    e = edge_index.shape[1]
    src = edge_index[0].astype(jnp.int32)
    dst = edge_index[1].astype(jnp.int32)

    nb = _num_batches(e)
    epad = _NC * _NS * nb * _EB
    src3 = jnp.concatenate(
        [src, jnp.zeros((epad - e,), jnp.int32)]).reshape(_NC * _NS, nb, _EB)
    dst3 = jnp.concatenate(
        [dst, n + jnp.arange(epad - e, dtype=jnp.int32) % _PAD]
    ).reshape(_NC * _NS, nb, _EB)

    zeros_d = jnp.zeros((n + _PAD, d_in), jnp.float32)
    ones_d = jnp.ones((_EB, d_in), jnp.float32)

    b1r = b1.reshape(1, -1)
    b2r = b2.reshape(1, -1)
    b3r = b3.reshape(1, -1)
    b_resr = b_res.reshape(1, -1)

    s1, degp = _make_seg_sum(n, e, d_in, 1, True)(
        features, src3, dst3, zeros_d, ones_d)
    deg16 = degp[:, :, :16]
    x1, res, n2a, n2b, n2c = _tc_layer1(features, s1, deg16, W_self1,
                                        W_neigh1, b1r, W_res, b_resr,
                                        W_neigh2)
    s2a, s2b, s2c = _make_seg_sum(n, e, 128, 3, False)(
        n2a, n2b, n2c, src3, dst3, zeros_d)
    x2, n3 = _tc_layer2(x1, s2a, s2b, s2c, deg16, W_self2, b2r, W_neigh3)
    (s3,) = _make_seg_sum(n, e, d_in, 1, False)(n3, src3, dst3, zeros_d)
    x3 = _tc_layer3(x2, s3, deg16, W_self3, b3r)
    return (x3, res)


# speedup vs baseline: 1.0331x; 1.0331x over previous
import functools

import jax
import jax.numpy as jnp
from jax import lax
from jax.experimental import pallas as pl
from jax.experimental.pallas import tpu as pltpu
from jax.experimental.pallas import tpu_sc as plsc

_NC = 2
_NS = 16
_EB = 80
_PAD = 128


def _elu(x):
    return jnp.where(x > 0, x, jnp.exp(jnp.minimum(x, 0.0)) - 1.0)


def _num_batches(e):
    nb = -(-e // (_NC * _NS * _EB))
    return -(-nb // 4) * 4


def _make_seg_sum(n, e, d, num_chunks, with_deg):
    assert d % 128 == 0
    nb = _num_batches(e)
    na = n + _PAD
    rbase = (n // (8 * _NS)) * 8
    zrem = na - _NS * rbase
    frem = n - _NS * rbase
    assert zrem % 8 == 0 and frem % 8 == 0 and frem >= 0

    nh = nb // 2
    mesh = plsc.VectorSubcoreMesh(core_axis_name="c", subcore_axis_name="s")
    out_type = [jax.ShapeDtypeStruct((_NC, n, d), jnp.float32)
                for _ in range(num_chunks + int(with_deg))]
    scratch = [
        pltpu.VMEM((nh, _EB), jnp.int32),
        pltpu.VMEM((nh, _EB), jnp.int32),
        pltpu.VMEM((_EB, d), jnp.float32),
        pltpu.VMEM((_EB, d), jnp.float32),
        pltpu.VMEM_SHARED((na, d), jnp.float32),
        pltpu.SemaphoreType.DMA,
        pltpu.SemaphoreType.DMA,
    ]

    @functools.partial(pl.kernel, out_type=out_type, mesh=mesh,
                       scratch_types=scratch)
    def k(*refs):
        it = iter(refs)
        tables = [next(it) for _ in range(num_chunks)]
        src_hbm, dst_hbm, zd_hbm = next(it), next(it), next(it)
        ones_hbm = next(it) if with_deg else None
        outs = [next(it) for _ in range(num_chunks + int(with_deg))]
        src_v, dst_v, rows0, rows1, acc_sh, sem0, sem1 = (
            next(it) for _ in range(7))

        c = lax.axis_index("c")
        s = lax.axis_index("s")
        wid = s * _NC + c
        row0 = pl.multiple_of(s * rbase, 8)

        def copy_rows(rem, get_src_dst):
            src, dst = get_src_dst(row0, rbase)
            pltpu.sync_copy(src, dst)
            if rem:
                @pl.when(s == _NS - 1)
                def _():
                    srct, dstt = get_src_dst(_NS * rbase, rem)
                    pltpu.sync_copy(srct, dstt)

        def load_idx(half):
            pltpu.sync_copy(src_hbm.at[wid, pl.ds(half * nh, nh)], src_v)
            pltpu.sync_copy(dst_hbm.at[wid, pl.ds(half * nh, nh)], dst_v)

        for ci in range(num_chunks + int(with_deg)):
            deg_pass = ci == num_chunks
            copy_rows(zrem, lambda r, m: (zd_hbm.at[pl.ds(r, m)],
                                          acc_sh.at[pl.ds(r, m)]))
            if deg_pass:
                pltpu.sync_copy(ones_hbm, rows0)
            plsc.subcore_barrier()

            for half in range(2):
                load_idx(half)
                if deg_pass:
                    def dbody(j, carry):
                        pltpu.async_copy(rows0, acc_sh.at[dst_v.at[j]],
                                         sem0, add=True)
                        return carry

                    lax.fori_loop(0, nh, dbody, 0)

                    def dwait(j, carry):
                        pltpu.make_async_copy(
                            rows0, acc_sh.at[dst_v.at[0]], sem0).wait()
                        return carry

                    lax.fori_loop(0, nh, dwait, 0)
                else:
                    tab = tables[ci]

                    def gbody(j, carry):
                        pltpu.async_copy(
                            tab.at[src_v.at[j]], rows0, sem0).wait()
                        pltpu.sync_copy(rows0, acc_sh.at[dst_v.at[j]],
                                        add=True)
                        return carry

                    lax.fori_loop(0, nh, gbody, 0)

            plsc.subcore_barrier()
            out_ref = outs[ci]
            copy_rows(frem, lambda r, m: (acc_sh.at[pl.ds(r, m)],
                                          out_ref.at[c, pl.ds(r, m)]))

    return k


_R = 2000


def _inv_deg(dg_ref):
    deg = dg_ref[0, :, 0:1] + dg_ref[1, :, 0:1]
    return 1.0 / jnp.maximum(deg, 1.0)


def _dot(a, b):
    return jnp.dot(a, b, preferred_element_type=jnp.float32)


def _row_blk(w):
    return pl.BlockSpec((_R, w), lambda i: (i, 0))


def _part_blk(w):
    return pl.BlockSpec((_NC, _R, w), lambda i: (0, i, 0))


def _full(a):
    return pl.BlockSpec(a.shape, lambda i: (0,) * a.ndim)


def _tc_layer1(features, s1, deg16, w_self1, w_neigh1, b1, w_res, b_res,
               w_neigh2):
    n, d_in = features.shape
    h1 = w_self1.shape[1]
    h2 = w_neigh2.shape[1]
    grid = n // _R

    def body(f_ref, s1_ref, dg_ref, ws1_ref, wn1_ref, b1_ref, wr_ref, br_ref,
             wn2_ref, x1_ref, res_ref, n2a_ref, n2b_ref, n2c_ref):
        inv = _inv_deg(dg_ref)
        m1 = (s1_ref[0] + s1_ref[1]) * inv
        f = f_ref[...]
        x1 = _elu(_dot(f, ws1_ref[...]) + _dot(m1, wn1_ref[...]) + b1_ref[...])
        x1_ref[...] = x1
        res_ref[...] = _elu(_dot(f, wr_ref[...]) + br_ref[...])
        n2 = _dot(x1, wn2_ref[...])
        n2a_ref[...] = n2[:, :128]
        n2b_ref[...] = n2[:, 128:256]
        n2c_ref[...] = jnp.concatenate(
            [n2[:, 256:], jnp.zeros((_R, 128 - (h2 - 256)), jnp.float32)],
            axis=1)

    return pl.pallas_call(
        body,
        grid=(grid,),
        in_specs=[_row_blk(d_in), _part_blk(d_in), _part_blk(16),
                  _full(w_self1), _full(w_neigh1), _full(b1), _full(w_res),
                  _full(b_res), _full(w_neigh2)],
        out_specs=[_row_blk(h1), _row_blk(d_in), _row_blk(128), _row_blk(128),
                   _row_blk(128)],
        out_shape=[jax.ShapeDtypeStruct((n, h1), jnp.float32),
                   jax.ShapeDtypeStruct((n, d_in), jnp.float32),
                   jax.ShapeDtypeStruct((n, 128), jnp.float32),
                   jax.ShapeDtypeStruct((n, 128), jnp.float32),
                   jax.ShapeDtypeStruct((n, 128), jnp.float32)],
    )(features, s1, deg16, w_self1, w_neigh1, b1, w_res, b_res, w_neigh2)


def _tc_layer2(x1, s2a, s2b, s2c, deg16, w_self2, b2, w_neigh3):
    n, h1 = x1.shape
    h2 = w_self2.shape[1]
    d_out = w_neigh3.shape[1]
    grid = n // _R

    def body(x1_ref, s2a_ref, s2b_ref, s2c_ref, dg_ref, ws2_ref, b2_ref,
             wn3_ref, x2_ref, n3_ref):
        inv = _inv_deg(dg_ref)
        m2 = jnp.concatenate(
            [(s2a_ref[0] + s2a_ref[1]) * inv,
             (s2b_ref[0] + s2b_ref[1]) * inv,
             ((s2c_ref[0] + s2c_ref[1]) * inv)[:, :h2 - 256]], axis=1)
        x2 = _elu(_dot(x1_ref[...], ws2_ref[...]) + m2 + b2_ref[...])
        x2_ref[...] = x2
        n3_ref[...] = _dot(x2, wn3_ref[...])

    return pl.pallas_call(
        body,
        grid=(grid,),
        in_specs=[_row_blk(h1), _part_blk(128), _part_blk(128),
                  _part_blk(128), _part_blk(16),
                  _full(w_self2), _full(b2), _full(w_neigh3)],
        out_specs=[_row_blk(h2), _row_blk(d_out)],
        out_shape=[jax.ShapeDtypeStruct((n, h2), jnp.float32),
                   jax.ShapeDtypeStruct((n, d_out), jnp.float32)],
    )(x1, s2a, s2b, s2c, deg16, w_self2, b2, w_neigh3)


def _tc_layer3(x2, s3, deg16, w_self3, b3):
    n, h2 = x2.shape
    d_out = w_self3.shape[1]
    grid = n // _R

    def body(x2_ref, s3_ref, dg_ref, ws3_ref, b3_ref, x3_ref):
        inv = _inv_deg(dg_ref)
        m3 = (s3_ref[0] + s3_ref[1]) * inv
        x3_ref[...] = _elu(_dot(x2_ref[...], ws3_ref[...]) + m3 + b3_ref[...])

    return pl.pallas_call(
        body,
        grid=(grid,),
        in_specs=[_row_blk(h2), _part_blk(d_out), _part_blk(16),
                  _full(w_self3), _full(b3)],
        out_specs=[_row_blk(d_out)],
        out_shape=[jax.ShapeDtypeStruct((n, d_out), jnp.float32)],
    )(x2, s3, deg16, w_self3, b3)[0]


def kernel(features, edge_index, W_self1, W_neigh1, b1, W_self2, W_neigh2,
           b2, W_self3, W_neigh3, b3, W_res, b_res):
    n, d_in = features.shape
    e = edge_index.shape[1]
    src = edge_index[0].astype(jnp.int32)
    dst = edge_index[1].astype(jnp.int32)

    nb = _num_batches(e)
    epad = _NC * _NS * nb * _EB
    src3 = jnp.concatenate(
        [src, jnp.zeros((epad - e,), jnp.int32)]).reshape(_NC * _NS, nb, _EB)
    dst3 = jnp.concatenate(
        [dst, n + jnp.arange(epad - e, dtype=jnp.int32) % _PAD]
    ).reshape(_NC * _NS, nb, _EB)

    zeros_d = jnp.zeros((n + _PAD, d_in), jnp.float32)
    ones_d = jnp.ones((_EB, d_in), jnp.float32)

    b1r = b1.reshape(1, -1)
    b2r = b2.reshape(1, -1)
    b3r = b3.reshape(1, -1)
    b_resr = b_res.reshape(1, -1)

    s1, degp = _make_seg_sum(n, e, d_in, 1, True)(
        features, src3, dst3, zeros_d, ones_d)
    deg16 = degp[:, :, :16]
    x1, res, n2a, n2b, n2c = _tc_layer1(features, s1, deg16, W_self1,
                                        W_neigh1, b1r, W_res, b_resr,
                                        W_neigh2)
    s2a, s2b, s2c = _make_seg_sum(n, e, 128, 3, False)(
        n2a, n2b, n2c, src3, dst3, zeros_d)
    x2, n3 = _tc_layer2(x1, s2a, s2b, s2c, deg16, W_self2, b2r, W_neigh3)
    (s3,) = _make_seg_sum(n, e, d_in, 1, False)(n3, src3, dst3, zeros_d)
    x3 = _tc_layer3(x2, s3, deg16, W_self3, b3r)
    return (x3, res)


# speedup vs baseline: 2.6241x; 2.5401x over previous
import functools

import jax
import jax.numpy as jnp
from jax import lax
from jax.experimental import pallas as pl
from jax.experimental.pallas import tpu as pltpu
from jax.experimental.pallas import tpu_sc as plsc

_NC = 2
_NS = 16
_EB = 80


def _elu(x):
    return jnp.where(x > 0, x, jnp.exp(jnp.minimum(x, 0.0)) - 1.0)


def _make_seg_sum(n, e, d, num_chunks, with_deg):
    assert d % 128 == 0
    epw = e // (_NC * _NS)
    assert epw % _EB == 0 and epw * _NC * _NS == e
    nb = epw // _EB
    rbase = (n // (8 * _NS)) * 8
    rrem = n - _NS * rbase
    assert rrem % 8 == 0 and rrem >= 0

    mesh = plsc.VectorSubcoreMesh(core_axis_name="c", subcore_axis_name="s")
    out_type = [jax.ShapeDtypeStruct((_NC, n, d), jnp.float32)
                for _ in range(num_chunks + int(with_deg))]
    scratch = [
        pltpu.VMEM((_EB,), jnp.int32),
        pltpu.VMEM((_EB,), jnp.int32),
        pltpu.VMEM((_EB,), jnp.int32),
        pltpu.VMEM((_EB,), jnp.int32),
        pltpu.VMEM((_EB, d), jnp.float32),
        pltpu.VMEM((_EB, d), jnp.float32),
        pltpu.VMEM_SHARED((n, d), jnp.float32),
        pltpu.SemaphoreType.DMA,
        pltpu.SemaphoreType.DMA,
    ]

    @functools.partial(pl.kernel, out_type=out_type, mesh=mesh,
                       scratch_types=scratch)
    def k(*refs):
        it = iter(refs)
        tables = [next(it) for _ in range(num_chunks)]
        src_hbm, dst_hbm, zd_hbm = next(it), next(it), next(it)
        ones_hbm = next(it) if with_deg else None
        outs = [next(it) for _ in range(num_chunks + int(with_deg))]
        (sidx0, sidx1, didx0, didx1, rows0, rows1, acc_sh, sem0,
         sem1) = (next(it) for _ in range(9))

        c = lax.axis_index("c")
        s = lax.axis_index("s")
        wid = s * _NC + c
        base0 = wid * epw
        row0 = pl.multiple_of(s * rbase, 8)

        def copy_rows(get_src_dst):
            src, dst = get_src_dst(row0, rbase)
            pltpu.sync_copy(src, dst)
            if rrem:
                @pl.when(s == _NS - 1)
                def _():
                    srct, dstt = get_src_dst(_NS * rbase, rrem)
                    pltpu.sync_copy(srct, dstt)

        def load_idx(j, sidx, didx, need_src=True):
            base = pl.multiple_of(base0 + j * _EB, 8)
            if need_src:
                pltpu.sync_copy(src_hbm.at[pl.ds(base, _EB)], sidx)
            pltpu.sync_copy(dst_hbm.at[pl.ds(base, _EB)], didx)

        slots = ((sidx0, didx0, rows0, sem0), (sidx1, didx1, rows1, sem1))

        for ci in range(num_chunks + int(with_deg)):
            deg_pass = ci == num_chunks
            copy_rows(lambda r, m: (zd_hbm.at[pl.ds(r, m)],
                                    acc_sh.at[pl.ds(r, m)]))
            if deg_pass:
                pltpu.sync_copy(ones_hbm, rows0)
            plsc.subcore_barrier()

            if deg_pass:
                for p, (_, didx, _, sem) in enumerate(slots):
                    load_idx(p, None, didx, need_src=False)
                    pltpu.async_copy(rows0, acc_sh.at[didx], sem, add=True)

                def dbody(i, carry):
                    j0 = 2 * i
                    for p, (_, didx, _, sem) in enumerate(slots):
                        j = j0 + p
                        pltpu.make_async_copy(
                            rows0, acc_sh.at[didx], sem).wait()

                        @pl.when(j + 2 < nb)
                        def _():
                            load_idx(j + 2, None, didx, need_src=False)
                            pltpu.async_copy(rows0, acc_sh.at[didx], sem,
                                             add=True)
                    return carry

                lax.fori_loop(0, nb // 2, dbody, 0)
                if nb % 2:
                    pltpu.make_async_copy(rows0, acc_sh.at[didx0],
                                          sem0).wait()
            else:
                tab = tables[ci]
                for p, (sidx, didx, rows, sem) in enumerate(slots):
                    load_idx(p, sidx, didx)
                    pltpu.async_copy(tab.at[sidx], rows, sem)

                def gbody(i, carry):
                    j0 = 2 * i
                    for p, (sidx, didx, rows, sem) in enumerate(slots):
                        j = j0 + p
                        pltpu.make_async_copy(tab.at[sidx], rows, sem).wait()
                        pltpu.sync_copy(rows, acc_sh.at[didx], add=True)

                        @pl.when(j + 2 < nb)
                        def _():
                            load_idx(j + 2, sidx, didx)
                            pltpu.async_copy(tab.at[sidx], rows, sem)
                    return carry

                lax.fori_loop(0, nb // 2, gbody, 0)
                if nb % 2:
                    pltpu.make_async_copy(tab.at[sidx0], rows0, sem0).wait()
                    pltpu.sync_copy(rows0, acc_sh.at[didx0], add=True)

            plsc.subcore_barrier()
            out_ref = outs[ci]
            copy_rows(lambda r, m: (acc_sh.at[pl.ds(r, m)],
                                    out_ref.at[c, pl.ds(r, m)]))

    return k


_R = 2000


def _inv_deg(dg_ref):
    deg = dg_ref[0, :, 0:1] + dg_ref[1, :, 0:1]
    return 1.0 / jnp.maximum(deg, 1.0)


def _dot(a, b):
    return jnp.dot(a, b, preferred_element_type=jnp.float32)


def _row_blk(w):
    return pl.BlockSpec((_R, w), lambda i: (i, 0))


def _part_blk(w):
    return pl.BlockSpec((_NC, _R, w), lambda i: (0, i, 0))


def _full(a):
    return pl.BlockSpec(a.shape, lambda i: (0,) * a.ndim)


def _tc_layer1(features, s1, deg16, w_self1, w_neigh1, b1, w_res, b_res,
               w_neigh2):
    n, d_in = features.shape
    h1 = w_self1.shape[1]
    h2 = w_neigh2.shape[1]
    grid = n // _R

    def body(f_ref, s1_ref, dg_ref, ws1_ref, wn1_ref, b1_ref, wr_ref, br_ref,
             wn2_ref, x1_ref, res_ref, n2a_ref, n2b_ref, n2c_ref):
        inv = _inv_deg(dg_ref)
        m1 = (s1_ref[0] + s1_ref[1]) * inv
        f = f_ref[...]
        x1 = _elu(_dot(f, ws1_ref[...]) + _dot(m1, wn1_ref[...]) + b1_ref[...])
        x1_ref[...] = x1
        res_ref[...] = _elu(_dot(f, wr_ref[...]) + br_ref[...])
        n2 = _dot(x1, wn2_ref[...])
        n2a_ref[...] = n2[:, :128]
        n2b_ref[...] = n2[:, 128:256]
        n2c_ref[...] = jnp.concatenate(
            [n2[:, 256:], jnp.zeros((_R, 128 - (h2 - 256)), jnp.float32)],
            axis=1)

    return pl.pallas_call(
        body,
        grid=(grid,),
        in_specs=[_row_blk(d_in), _part_blk(d_in), _part_blk(16),
                  _full(w_self1), _full(w_neigh1), _full(b1), _full(w_res),
                  _full(b_res), _full(w_neigh2)],
        out_specs=[_row_blk(h1), _row_blk(d_in), _row_blk(128), _row_blk(128),
                   _row_blk(128)],
        out_shape=[jax.ShapeDtypeStruct((n, h1), jnp.float32),
                   jax.ShapeDtypeStruct((n, d_in), jnp.float32),
                   jax.ShapeDtypeStruct((n, 128), jnp.float32),
                   jax.ShapeDtypeStruct((n, 128), jnp.float32),
                   jax.ShapeDtypeStruct((n, 128), jnp.float32)],
    )(features, s1, deg16, w_self1, w_neigh1, b1, w_res, b_res, w_neigh2)


def _tc_layer2(x1, s2a, s2b, s2c, deg16, w_self2, b2, w_neigh3):
    n, h1 = x1.shape
    h2 = w_self2.shape[1]
    d_out = w_neigh3.shape[1]
    grid = n // _R

    def body(x1_ref, s2a_ref, s2b_ref, s2c_ref, dg_ref, ws2_ref, b2_ref,
             wn3_ref, x2_ref, n3_ref):
        inv = _inv_deg(dg_ref)
        m2 = jnp.concatenate(
            [(s2a_ref[0] + s2a_ref[1]) * inv,
             (s2b_ref[0] + s2b_ref[1]) * inv,
             ((s2c_ref[0] + s2c_ref[1]) * inv)[:, :h2 - 256]], axis=1)
        x2 = _elu(_dot(x1_ref[...], ws2_ref[...]) + m2 + b2_ref[...])
        x2_ref[...] = x2
        n3_ref[...] = _dot(x2, wn3_ref[...])

    return pl.pallas_call(
        body,
        grid=(grid,),
        in_specs=[_row_blk(h1), _part_blk(128), _part_blk(128),
                  _part_blk(128), _part_blk(16),
                  _full(w_self2), _full(b2), _full(w_neigh3)],
        out_specs=[_row_blk(h2), _row_blk(d_out)],
        out_shape=[jax.ShapeDtypeStruct((n, h2), jnp.float32),
                   jax.ShapeDtypeStruct((n, d_out), jnp.float32)],
    )(x1, s2a, s2b, s2c, deg16, w_self2, b2, w_neigh3)


def _tc_layer3(x2, s3, deg16, w_self3, b3):
    n, h2 = x2.shape
    d_out = w_self3.shape[1]
    grid = n // _R

    def body(x2_ref, s3_ref, dg_ref, ws3_ref, b3_ref, x3_ref):
        inv = _inv_deg(dg_ref)
        m3 = (s3_ref[0] + s3_ref[1]) * inv
        x3_ref[...] = _elu(_dot(x2_ref[...], ws3_ref[...]) + m3 + b3_ref[...])

    return pl.pallas_call(
        body,
        grid=(grid,),
        in_specs=[_row_blk(h2), _part_blk(d_out), _part_blk(16),
                  _full(w_self3), _full(b3)],
        out_specs=[_row_blk(d_out)],
        out_shape=[jax.ShapeDtypeStruct((n, d_out), jnp.float32)],
    )(x2, s3, deg16, w_self3, b3)[0]


def kernel(features, edge_index, W_self1, W_neigh1, b1, W_self2, W_neigh2,
           b2, W_self3, W_neigh3, b3, W_res, b_res):
    n, d_in = features.shape
    e = edge_index.shape[1]
    src = edge_index[0].astype(jnp.int32)
    dst = edge_index[1].astype(jnp.int32)

    zeros_d = jnp.zeros((n, d_in), jnp.float32)
    ones_d = jnp.ones((_EB, d_in), jnp.float32)

    b1r = b1.reshape(1, -1)
    b2r = b2.reshape(1, -1)
    b3r = b3.reshape(1, -1)
    b_resr = b_res.reshape(1, -1)

    s1, degp = _make_seg_sum(n, e, d_in, 1, True)(
        features, src, dst, zeros_d, ones_d)
    deg16 = degp[:, :, :16]
    x1, res, n2a, n2b, n2c = _tc_layer1(features, s1, deg16, W_self1,
                                        W_neigh1, b1r, W_res, b_resr,
                                        W_neigh2)
    s2a, s2b, s2c = _make_seg_sum(n, e, 128, 3, False)(
        n2a, n2b, n2c, src, dst, zeros_d)
    x2, n3 = _tc_layer2(x1, s2a, s2b, s2c, deg16, W_self2, b2r, W_neigh3)
    (s3,) = _make_seg_sum(n, e, d_in, 1, False)(n3, src, dst, zeros_d)
    x3 = _tc_layer3(x2, s3, deg16, W_self3, b3r)
    return (x3, res)


# speedup vs baseline: 3.7624x; 1.4338x over previous
import functools

import jax
import jax.numpy as jnp
from jax import lax
from jax.experimental import pallas as pl
from jax.experimental.pallas import tpu as pltpu
from jax.experimental.pallas import tpu_sc as plsc

_NC = 2
_NS = 16
_EB = 80


def _elu(x):
    return jnp.where(x > 0, x, jnp.exp(jnp.minimum(x, 0.0)) - 1.0)


def _make_seg_sum(n, e, d, num_chunks, with_deg):
    assert d % 128 == 0
    epw = e // (_NC * _NS)
    assert epw % _EB == 0 and epw * _NC * _NS == e
    nb = epw // _EB
    rbase = (n // (8 * _NS)) * 8
    rrem = n - _NS * rbase
    assert rrem % 8 == 0 and rrem >= 0

    mesh = plsc.VectorSubcoreMesh(core_axis_name="c", subcore_axis_name="s")
    out_type = [jax.ShapeDtypeStruct((_NC, n, d), jnp.float32)
                for _ in range(num_chunks + int(with_deg))]
    nslots = 4
    scratch = (
        [pltpu.VMEM((_EB,), jnp.int32) for _ in range(nslots)]
        + [pltpu.VMEM((_EB,), jnp.int32) for _ in range(nslots)]
        + [pltpu.VMEM((_EB, d), jnp.float32) for _ in range(nslots)]
        + [pltpu.VMEM_SHARED((n, d), jnp.float32)]
        + [pltpu.SemaphoreType.DMA for _ in range(nslots)]
        + [pltpu.SemaphoreType.DMA for _ in range(nslots)]
    )

    @functools.partial(pl.kernel, out_type=out_type, mesh=mesh,
                       scratch_types=scratch)
    def k(*refs):
        it = iter(refs)
        tables = [next(it) for _ in range(num_chunks)]
        src_hbm, dst_hbm, zd_hbm = next(it), next(it), next(it)
        ones_hbm = next(it) if with_deg else None
        outs = [next(it) for _ in range(num_chunks + int(with_deg))]
        sidx = [next(it) for _ in range(nslots)]
        didx = [next(it) for _ in range(nslots)]
        rows = [next(it) for _ in range(nslots)]
        acc_sh = next(it)
        gsem = [next(it) for _ in range(nslots)]
        isem = [next(it) for _ in range(nslots)]

        c = lax.axis_index("c")
        s = lax.axis_index("s")
        wid = s * _NC + c
        base0 = wid * epw
        row0 = pl.multiple_of(s * rbase, 8)

        def copy_rows(get_src_dst):
            src, dst = get_src_dst(row0, rbase)
            pltpu.sync_copy(src, dst)
            if rrem:
                @pl.when(s == _NS - 1)
                def _():
                    srct, dstt = get_src_dst(_NS * rbase, rrem)
                    pltpu.sync_copy(srct, dstt)

        def issue_idx(j, p, need_src):
            base = pl.multiple_of(base0 + j * _EB, 8)
            if need_src:
                pltpu.async_copy(src_hbm.at[pl.ds(base, _EB)], sidx[p],
                                 isem[p])
            pltpu.async_copy(dst_hbm.at[pl.ds(base, _EB)], didx[p], isem[p])

        def wait_idx(p, need_src):
            if need_src:
                pltpu.make_async_copy(src_hbm.at[pl.ds(0, _EB)], sidx[p],
                                      isem[p]).wait()
            pltpu.make_async_copy(dst_hbm.at[pl.ds(0, _EB)], didx[p],
                                  isem[p]).wait()

        for ci in range(num_chunks + int(with_deg)):
            deg_pass = ci == num_chunks
            copy_rows(lambda r, m: (zd_hbm.at[pl.ds(r, m)],
                                    acc_sh.at[pl.ds(r, m)]))
            if deg_pass:
                pltpu.sync_copy(ones_hbm, rows[0])
            plsc.subcore_barrier()

            if deg_pass:
                for p in range(nslots):
                    issue_idx(p, p, False)
                for p in range(nslots):
                    wait_idx(p, False)
                    pltpu.async_copy(rows[0], acc_sh.at[didx[p]], gsem[p],
                                     add=True)

                def dbody(i, carry):
                    j0 = nslots * i
                    for p in range(nslots):
                        j = j0 + p
                        pltpu.make_async_copy(
                            rows[0], acc_sh.at[didx[p]], gsem[p]).wait()

                        @pl.when(j + nslots < nb)
                        def _():
                            issue_idx(j + nslots, p, False)
                            wait_idx(p, False)
                            pltpu.async_copy(rows[0], acc_sh.at[didx[p]],
                                             gsem[p], add=True)
                    return carry

                lax.fori_loop(0, nb // nslots, dbody, 0)
                for j in range(nb - nb % nslots, nb):
                    pltpu.make_async_copy(
                        rows[0], acc_sh.at[didx[j % nslots]],
                        gsem[j % nslots]).wait()
            else:
                tab = tables[ci]
                for p in range(nslots):
                    issue_idx(p, p, True)
                for j in (0, 1):
                    wait_idx(j, True)
                    pltpu.async_copy(tab.at[sidx[j]], rows[j], gsem[j])

                def gbody(i, carry):
                    j0 = nslots * i
                    for p in range(nslots):
                        j = j0 + p
                        q = (p + 2) % nslots
                        pltpu.make_async_copy(tab.at[sidx[p]], rows[p],
                                              gsem[p]).wait()
                        pltpu.sync_copy(rows[p], acc_sh.at[didx[p]],
                                        add=True)

                        @pl.when(j + nslots < nb)
                        def _():
                            issue_idx(j + nslots, p, True)

                        @pl.when(j + 2 < nb)
                        def _():
                            wait_idx(q, True)
                            pltpu.async_copy(tab.at[sidx[q]], rows[q],
                                             gsem[q])
                    return carry

                lax.fori_loop(0, nb // nslots, gbody, 0)
                for j in range(nb - nb % nslots, nb):
                    p = j % nslots
                    pltpu.make_async_copy(tab.at[sidx[p]], rows[p],
                                          gsem[p]).wait()
                    pltpu.sync_copy(rows[p], acc_sh.at[didx[p]], add=True)

            plsc.subcore_barrier()
            out_ref = outs[ci]
            copy_rows(lambda r, m: (acc_sh.at[pl.ds(r, m)],
                                    out_ref.at[c, pl.ds(r, m)]))

    return k


_R = 2000


def _inv_deg(dg_ref):
    deg = dg_ref[0, :, 0:1] + dg_ref[1, :, 0:1]
    return 1.0 / jnp.maximum(deg, 1.0)


def _dot(a, b):
    return jnp.dot(a, b, preferred_element_type=jnp.float32)


def _row_blk(w):
    return pl.BlockSpec((_R, w), lambda i: (i, 0))


def _part_blk(w):
    return pl.BlockSpec((_NC, _R, w), lambda i: (0, i, 0))


def _full(a):
    return pl.BlockSpec(a.shape, lambda i: (0,) * a.ndim)


def _tc_layer1(features, s1, deg16, w_self1, w_neigh1, b1, w_res, b_res,
               w_neigh2):
    n, d_in = features.shape
    h1 = w_self1.shape[1]
    h2 = w_neigh2.shape[1]
    grid = n // _R

    def body(f_ref, s1_ref, dg_ref, ws1_ref, wn1_ref, b1_ref, wr_ref, br_ref,
             wn2_ref, x1_ref, res_ref, n2a_ref, n2b_ref, n2c_ref):
        inv = _inv_deg(dg_ref)
        m1 = (s1_ref[0] + s1_ref[1]) * inv
        f = f_ref[...]
        x1 = _elu(_dot(f, ws1_ref[...]) + _dot(m1, wn1_ref[...]) + b1_ref[...])
        x1_ref[...] = x1
        res_ref[...] = _elu(_dot(f, wr_ref[...]) + br_ref[...])
        n2 = _dot(x1, wn2_ref[...])
        n2a_ref[...] = n2[:, :128]
        n2b_ref[...] = n2[:, 128:256]
        n2c_ref[...] = jnp.concatenate(
            [n2[:, 256:], jnp.zeros((_R, 128 - (h2 - 256)), jnp.float32)],
            axis=1)

    return pl.pallas_call(
        body,
        grid=(grid,),
        in_specs=[_row_blk(d_in), _part_blk(d_in), _part_blk(16),
                  _full(w_self1), _full(w_neigh1), _full(b1), _full(w_res),
                  _full(b_res), _full(w_neigh2)],
        out_specs=[_row_blk(h1), _row_blk(d_in), _row_blk(128), _row_blk(128),
                   _row_blk(128)],
        out_shape=[jax.ShapeDtypeStruct((n, h1), jnp.float32),
                   jax.ShapeDtypeStruct((n, d_in), jnp.float32),
                   jax.ShapeDtypeStruct((n, 128), jnp.float32),
                   jax.ShapeDtypeStruct((n, 128), jnp.float32),
                   jax.ShapeDtypeStruct((n, 128), jnp.float32)],
    )(features, s1, deg16, w_self1, w_neigh1, b1, w_res, b_res, w_neigh2)


def _tc_layer2(x1, s2a, s2b, s2c, deg16, w_self2, b2, w_neigh3):
    n, h1 = x1.shape
    h2 = w_self2.shape[1]
    d_out = w_neigh3.shape[1]
    grid = n // _R

    def body(x1_ref, s2a_ref, s2b_ref, s2c_ref, dg_ref, ws2_ref, b2_ref,
             wn3_ref, x2_ref, n3_ref):
        inv = _inv_deg(dg_ref)
        m2 = jnp.concatenate(
            [(s2a_ref[0] + s2a_ref[1]) * inv,
             (s2b_ref[0] + s2b_ref[1]) * inv,
             ((s2c_ref[0] + s2c_ref[1]) * inv)[:, :h2 - 256]], axis=1)
        x2 = _elu(_dot(x1_ref[...], ws2_ref[...]) + m2 + b2_ref[...])
        x2_ref[...] = x2
        n3_ref[...] = _dot(x2, wn3_ref[...])

    return pl.pallas_call(
        body,
        grid=(grid,),
        in_specs=[_row_blk(h1), _part_blk(128), _part_blk(128),
                  _part_blk(128), _part_blk(16),
                  _full(w_self2), _full(b2), _full(w_neigh3)],
        out_specs=[_row_blk(h2), _row_blk(d_out)],
        out_shape=[jax.ShapeDtypeStruct((n, h2), jnp.float32),
                   jax.ShapeDtypeStruct((n, d_out), jnp.float32)],
    )(x1, s2a, s2b, s2c, deg16, w_self2, b2, w_neigh3)


def _tc_layer3(x2, s3, deg16, w_self3, b3):
    n, h2 = x2.shape
    d_out = w_self3.shape[1]
    grid = n // _R

    def body(x2_ref, s3_ref, dg_ref, ws3_ref, b3_ref, x3_ref):
        inv = _inv_deg(dg_ref)
        m3 = (s3_ref[0] + s3_ref[1]) * inv
        x3_ref[...] = _elu(_dot(x2_ref[...], ws3_ref[...]) + m3 + b3_ref[...])

    return pl.pallas_call(
        body,
        grid=(grid,),
        in_specs=[_row_blk(h2), _part_blk(d_out), _part_blk(16),
                  _full(w_self3), _full(b3)],
        out_specs=[_row_blk(d_out)],
        out_shape=[jax.ShapeDtypeStruct((n, d_out), jnp.float32)],
    )(x2, s3, deg16, w_self3, b3)[0]


def kernel(features, edge_index, W_self1, W_neigh1, b1, W_self2, W_neigh2,
           b2, W_self3, W_neigh3, b3, W_res, b_res):
    n, d_in = features.shape
    e = edge_index.shape[1]
    src = edge_index[0].astype(jnp.int32)
    dst = edge_index[1].astype(jnp.int32)

    zeros_d = jnp.zeros((n, d_in), jnp.float32)
    ones_d = jnp.ones((_EB, d_in), jnp.float32)

    b1r = b1.reshape(1, -1)
    b2r = b2.reshape(1, -1)
    b3r = b3.reshape(1, -1)
    b_resr = b_res.reshape(1, -1)

    s1, degp = _make_seg_sum(n, e, d_in, 1, True)(
        features, src, dst, zeros_d, ones_d)
    deg16 = degp[:, :, :16]
    x1, res, n2a, n2b, n2c = _tc_layer1(features, s1, deg16, W_self1,
                                        W_neigh1, b1r, W_res, b_resr,
                                        W_neigh2)
    s2a, s2b, s2c = _make_seg_sum(n, e, 128, 3, False)(
        n2a, n2b, n2c, src, dst, zeros_d)
    x2, n3 = _tc_layer2(x1, s2a, s2b, s2c, deg16, W_self2, b2r, W_neigh3)
    (s3,) = _make_seg_sum(n, e, d_in, 1, False)(n3, src, dst, zeros_d)
    x3 = _tc_layer3(x2, s3, deg16, W_self3, b3r)
    return (x3, res)


# speedup vs baseline: 3.8413x; 1.0210x over previous
import functools

import jax
import jax.numpy as jnp
from jax import lax
from jax.experimental import pallas as pl
from jax.experimental.pallas import tpu as pltpu
from jax.experimental.pallas import tpu_sc as plsc

_NC = 2
_NS = 16
_EB = 80


def _elu(x):
    return jnp.where(x > 0, x, jnp.exp(jnp.minimum(x, 0.0)) - 1.0)


def _make_seg_sum(n, e, d, num_chunks, with_deg):
    assert d % 128 == 0
    epw = e // (_NC * _NS)
    assert epw % _EB == 0 and epw * _NC * _NS == e
    nb = epw // _EB
    rbase = (n // (8 * _NS)) * 8
    rrem = n - _NS * rbase
    assert rrem % 8 == 0 and rrem >= 0

    mesh = plsc.VectorSubcoreMesh(core_axis_name="c", subcore_axis_name="s")
    out_type = [jax.ShapeDtypeStruct((_NC, n, d), jnp.float32)
                for _ in range(num_chunks + int(with_deg))]
    nslots = 4
    scratch = (
        [pltpu.VMEM((_EB,), jnp.int32) for _ in range(nslots)]
        + [pltpu.VMEM((_EB,), jnp.int32) for _ in range(nslots)]
        + [pltpu.VMEM((_EB, d), jnp.float32) for _ in range(nslots)]
        + [pltpu.VMEM_SHARED((n, d), jnp.float32)]
        + [pltpu.SemaphoreType.DMA for _ in range(nslots)]
        + [pltpu.SemaphoreType.DMA for _ in range(nslots)]
        + [pltpu.SemaphoreType.DMA for _ in range(nslots)]
        + [pltpu.SemaphoreType.DMA for _ in range(nslots)]
    )

    @functools.partial(pl.kernel, out_type=out_type, mesh=mesh,
                       scratch_types=scratch)
    def k(*refs):
        it = iter(refs)
        tables = [next(it) for _ in range(num_chunks)]
        src_hbm, dst_hbm, zd_hbm = next(it), next(it), next(it)
        ones_hbm = next(it) if with_deg else None
        outs = [next(it) for _ in range(num_chunks + int(with_deg))]
        sidx = [next(it) for _ in range(nslots)]
        didx = [next(it) for _ in range(nslots)]
        rows = [next(it) for _ in range(nslots)]
        acc_sh = next(it)
        gsem = [next(it) for _ in range(nslots)]
        isem = [next(it) for _ in range(nslots)]
        jsem = [next(it) for _ in range(nslots)]
        asem = [next(it) for _ in range(nslots)]

        c = lax.axis_index("c")
        s = lax.axis_index("s")
        wid = s * _NC + c
        base0 = wid * epw
        row0 = pl.multiple_of(s * rbase, 8)

        def copy_rows(get_src_dst):
            src, dst = get_src_dst(row0, rbase)
            pltpu.sync_copy(src, dst)
            if rrem:
                @pl.when(s == _NS - 1)
                def _():
                    srct, dstt = get_src_dst(_NS * rbase, rrem)
                    pltpu.sync_copy(srct, dstt)

        def issue_sidx(j, p):
            base = pl.multiple_of(base0 + j * _EB, 8)
            pltpu.async_copy(src_hbm.at[pl.ds(base, _EB)], sidx[p], isem[p])

        def wait_sidx(p):
            pltpu.make_async_copy(src_hbm.at[pl.ds(0, _EB)], sidx[p],
                                  isem[p]).wait()

        def issue_didx(j, p):
            base = pl.multiple_of(base0 + j * _EB, 8)
            pltpu.async_copy(dst_hbm.at[pl.ds(base, _EB)], didx[p], jsem[p])

        def wait_didx(p):
            pltpu.make_async_copy(dst_hbm.at[pl.ds(0, _EB)], didx[p],
                                  jsem[p]).wait()

        for ci in range(num_chunks + int(with_deg)):
            deg_pass = ci == num_chunks
            copy_rows(lambda r, m: (zd_hbm.at[pl.ds(r, m)],
                                    acc_sh.at[pl.ds(r, m)]))
            if deg_pass:
                pltpu.sync_copy(ones_hbm, rows[0])
            plsc.subcore_barrier()

            if deg_pass:
                for p in range(nslots):
                    issue_didx(p, p)
                for p in range(nslots):
                    wait_didx(p)
                    pltpu.async_copy(rows[0], acc_sh.at[didx[p]], asem[p],
                                     add=True)

                def dbody(i, carry):
                    j0 = nslots * i
                    for p in range(nslots):
                        j = j0 + p
                        pltpu.make_async_copy(
                            rows[0], acc_sh.at[didx[p]], asem[p]).wait()

                        @pl.when(j + nslots < nb)
                        def _():
                            issue_didx(j + nslots, p)
                            wait_didx(p)
                            pltpu.async_copy(rows[0], acc_sh.at[didx[p]],
                                             asem[p], add=True)
                    return carry

                lax.fori_loop(0, nb // nslots, dbody, 0)
                for j in range(nb - nb % nslots, nb):
                    pltpu.make_async_copy(
                        rows[0], acc_sh.at[didx[j % nslots]],
                        asem[j % nslots]).wait()
            else:
                tab = tables[ci]
                for p in range(nslots):
                    issue_sidx(p, p)
                for j in (0, 1):
                    issue_didx(j, j)
                    wait_sidx(j)
                    pltpu.async_copy(tab.at[sidx[j]], rows[j], gsem[j])

                def visit(j, p, q, tail):
                    pltpu.make_async_copy(tab.at[sidx[p]], rows[p],
                                          gsem[p]).wait()
                    wait_didx(p)
                    pltpu.async_copy(rows[p], acc_sh.at[didx[p]], asem[p],
                                     add=True)
                    if tail:
                        return

                    @pl.when(j + nslots < nb)
                    def _():
                        issue_sidx(j + nslots, p)

                    @pl.when(j + 2 < nb)
                    def _():
                        @pl.when(j >= 2)
                        def _():
                            pltpu.make_async_copy(
                                rows[q], acc_sh.at[didx[q]], asem[q]).wait()
                        issue_didx(j + 2, q)
                        wait_sidx(q)
                        pltpu.async_copy(tab.at[sidx[q]], rows[q], gsem[q])

                def gbody(i, carry):
                    j0 = nslots * i
                    for p in range(nslots):
                        visit(j0 + p, p, (p + 2) % nslots, False)
                    return carry

                lax.fori_loop(0, nb // nslots, gbody, 0)
                for j in range(nb - nb % nslots, nb):
                    visit(j, j % nslots, (j + 2) % nslots, True)
                for j in range(max(0, nb - 4), nb):
                    pltpu.make_async_copy(rows[j % nslots],
                                          acc_sh.at[didx[j % nslots]],
                                          asem[j % nslots]).wait()

            plsc.subcore_barrier()
            out_ref = outs[ci]
            copy_rows(lambda r, m: (acc_sh.at[pl.ds(r, m)],
                                    out_ref.at[c, pl.ds(r, m)]))

    return k


_R = 2000


def _inv_deg(dg_ref):
    deg = dg_ref[0, :, 0:1] + dg_ref[1, :, 0:1]
    return 1.0 / jnp.maximum(deg, 1.0)


def _dot(a, b):
    return jnp.dot(a, b, preferred_element_type=jnp.float32)


def _row_blk(w):
    return pl.BlockSpec((_R, w), lambda i: (i, 0))


def _part_blk(w):
    return pl.BlockSpec((_NC, _R, w), lambda i: (0, i, 0))


def _full(a):
    return pl.BlockSpec(a.shape, lambda i: (0,) * a.ndim)


def _tc_layer1(features, s1, deg16, w_self1, w_neigh1, b1, w_res, b_res,
               w_neigh2):
    n, d_in = features.shape
    h1 = w_self1.shape[1]
    h2 = w_neigh2.shape[1]
    grid = n // _R

    def body(f_ref, s1_ref, dg_ref, ws1_ref, wn1_ref, b1_ref, wr_ref, br_ref,
             wn2_ref, x1_ref, res_ref, n2a_ref, n2b_ref, n2c_ref):
        inv = _inv_deg(dg_ref)
        m1 = (s1_ref[0] + s1_ref[1]) * inv
        f = f_ref[...]
        x1 = _elu(_dot(f, ws1_ref[...]) + _dot(m1, wn1_ref[...]) + b1_ref[...])
        x1_ref[...] = x1
        res_ref[...] = _elu(_dot(f, wr_ref[...]) + br_ref[...])
        n2 = _dot(x1, wn2_ref[...])
        n2a_ref[...] = n2[:, :128]
        n2b_ref[...] = n2[:, 128:256]
        n2c_ref[...] = jnp.concatenate(
            [n2[:, 256:], jnp.zeros((_R, 128 - (h2 - 256)), jnp.float32)],
            axis=1)

    return pl.pallas_call(
        body,
        grid=(grid,),
        in_specs=[_row_blk(d_in), _part_blk(d_in), _part_blk(16),
                  _full(w_self1), _full(w_neigh1), _full(b1), _full(w_res),
                  _full(b_res), _full(w_neigh2)],
        out_specs=[_row_blk(h1), _row_blk(d_in), _row_blk(128), _row_blk(128),
                   _row_blk(128)],
        out_shape=[jax.ShapeDtypeStruct((n, h1), jnp.float32),
                   jax.ShapeDtypeStruct((n, d_in), jnp.float32),
                   jax.ShapeDtypeStruct((n, 128), jnp.float32),
                   jax.ShapeDtypeStruct((n, 128), jnp.float32),
                   jax.ShapeDtypeStruct((n, 128), jnp.float32)],
    )(features, s1, deg16, w_self1, w_neigh1, b1, w_res, b_res, w_neigh2)


def _tc_layer2(x1, s2a, s2b, s2c, deg16, w_self2, b2, w_neigh3):
    n, h1 = x1.shape
    h2 = w_self2.shape[1]
    d_out = w_neigh3.shape[1]
    grid = n // _R

    def body(x1_ref, s2a_ref, s2b_ref, s2c_ref, dg_ref, ws2_ref, b2_ref,
             wn3_ref, x2_ref, n3_ref):
        inv = _inv_deg(dg_ref)
        m2 = jnp.concatenate(
            [(s2a_ref[0] + s2a_ref[1]) * inv,
             (s2b_ref[0] + s2b_ref[1]) * inv,
             ((s2c_ref[0] + s2c_ref[1]) * inv)[:, :h2 - 256]], axis=1)
        x2 = _elu(_dot(x1_ref[...], ws2_ref[...]) + m2 + b2_ref[...])
        x2_ref[...] = x2
        n3_ref[...] = _dot(x2, wn3_ref[...])

    return pl.pallas_call(
        body,
        grid=(grid,),
        in_specs=[_row_blk(h1), _part_blk(128), _part_blk(128),
                  _part_blk(128), _part_blk(16),
                  _full(w_self2), _full(b2), _full(w_neigh3)],
        out_specs=[_row_blk(h2), _row_blk(d_out)],
        out_shape=[jax.ShapeDtypeStruct((n, h2), jnp.float32),
                   jax.ShapeDtypeStruct((n, d_out), jnp.float32)],
    )(x1, s2a, s2b, s2c, deg16, w_self2, b2, w_neigh3)


def _tc_layer3(x2, s3, deg16, w_self3, b3):
    n, h2 = x2.shape
    d_out = w_self3.shape[1]
    grid = n // _R

    def body(x2_ref, s3_ref, dg_ref, ws3_ref, b3_ref, x3_ref):
        inv = _inv_deg(dg_ref)
        m3 = (s3_ref[0] + s3_ref[1]) * inv
        x3_ref[...] = _elu(_dot(x2_ref[...], ws3_ref[...]) + m3 + b3_ref[...])

    return pl.pallas_call(
        body,
        grid=(grid,),
        in_specs=[_row_blk(h2), _part_blk(d_out), _part_blk(16),
                  _full(w_self3), _full(b3)],
        out_specs=[_row_blk(d_out)],
        out_shape=[jax.ShapeDtypeStruct((n, d_out), jnp.float32)],
    )(x2, s3, deg16, w_self3, b3)[0]


def kernel(features, edge_index, W_self1, W_neigh1, b1, W_self2, W_neigh2,
           b2, W_self3, W_neigh3, b3, W_res, b_res):
    n, d_in = features.shape
    e = edge_index.shape[1]
    src = edge_index[0].astype(jnp.int32)
    dst = edge_index[1].astype(jnp.int32)

    zeros_d = jnp.zeros((n, d_in), jnp.float32)
    ones_d = jnp.ones((_EB, d_in), jnp.float32)

    b1r = b1.reshape(1, -1)
    b2r = b2.reshape(1, -1)
    b3r = b3.reshape(1, -1)
    b_resr = b_res.reshape(1, -1)

    s1, degp = _make_seg_sum(n, e, d_in, 1, True)(
        features, src, dst, zeros_d, ones_d)
    deg16 = degp[:, :, :16]
    x1, res, n2a, n2b, n2c = _tc_layer1(features, s1, deg16, W_self1,
                                        W_neigh1, b1r, W_res, b_resr,
                                        W_neigh2)
    s2a, s2b, s2c = _make_seg_sum(n, e, 128, 3, False)(
        n2a, n2b, n2c, src, dst, zeros_d)
    x2, n3 = _tc_layer2(x1, s2a, s2b, s2c, deg16, W_self2, b2r, W_neigh3)
    (s3,) = _make_seg_sum(n, e, d_in, 1, False)(n3, src, dst, zeros_d)
    x3 = _tc_layer3(x2, s3, deg16, W_self3, b3r)
    return (x3, res)
